# baseline (speedup 1.0000x reference)
import functools

import jax
import jax.numpy as jnp
from jax import lax
from jax.experimental import pallas as pl
from jax.experimental.pallas import tpu as pltpu
from jax.experimental.pallas import tpu_sc as plsc

_NUM_CORES = 2
_NUM_SUBCORES = 16
_LANES = 16


def _pack_pos(pos_table):
    S, D = pos_table.shape
    pb = pos_table.astype(jnp.bfloat16).reshape(S, D // 32, 2, _LANES)
    u = lax.bitcast_convert_type(pb, jnp.uint16).astype(jnp.uint32)
    words = (u[:, :, 1, :] << 16) | u[:, :, 0, :]
    return lax.bitcast_convert_type(words, jnp.int32).reshape(S, D // 2)


def kernel(x, tkn_table, pos_table):
    B, S = x.shape
    V, D = tkn_table.shape
    NW = _NUM_CORES * _NUM_SUBCORES
    C = S // NW
    assert S % NW == 0 and D % 32 == 0

    x = x.astype(jnp.int32)
    pos_pk = _pack_pos(pos_table)

    mesh = plsc.VectorSubcoreMesh(core_axis_name="c", subcore_axis_name="s")

    @functools.partial(
        pl.kernel,
        mesh=mesh,
        out_type=jax.ShapeDtypeStruct((B, S, D), jnp.float32),
        scratch_types=[
            pltpu.VMEM((B, C), jnp.int32),
            pltpu.VMEM((C, D // 2), jnp.int32),
            pltpu.VMEM((C, D), jnp.float32),
            pltpu.VMEM((C, D), jnp.float32),
            pltpu.SemaphoreType.DMA,
            pltpu.SemaphoreType.DMA,
            pltpu.SemaphoreType.DMA,
        ],
    )
    def emb(x_hbm, tkn_hbm, pos_hbm, out_hbm, idx_v, pos_v, t0, t1, gsem, w0, w1):
        bufs = [t0, t1]
        wsems = [w0, w1]
        wid = lax.axis_index("s") * _NUM_CORES + lax.axis_index("c")
        s0 = wid * C
        for b in range(B):
            pltpu.sync_copy(x_hbm.at[b, pl.ds(s0, C)], idx_v.at[b])
        pltpu.sync_copy(pos_hbm.at[pl.ds(s0, C)], pos_v)
        pltpu.async_copy(tkn_hbm.at[idx_v.at[0]], bufs[0], gsem).wait()

        writes = [None] * B
        for b in range(B):
            cur = bufs[b % 2]
            if b >= 1:
                writes[b - 1].wait()

            def row_body(r, carry, _cur=cur):
                for c2 in range(D // 32):
                    w = pos_v[r, pl.ds(c2 * _LANES, _LANES)]
                    lo = lax.bitcast_convert_type(
                        lax.shift_left(w, 16), jnp.float32
                    )
                    hi = lax.bitcast_convert_type(
                        lax.bitwise_and(w, jnp.int32(-65536)), jnp.float32
                    )
                    sl0 = pl.ds(c2 * 32, _LANES)
                    sl1 = pl.ds(c2 * 32 + _LANES, _LANES)
                    _cur[r, sl0] = _cur[r, sl0] + lo
                    _cur[r, sl1] = _cur[r, sl1] + hi
                return carry

            lax.fori_loop(0, C, row_body, 0)
            writes[b] = pltpu.async_copy(
                cur, out_hbm.at[b, pl.ds(s0, C)], wsems[b % 2]
            )
            if b + 1 < B:
                pltpu.async_copy(
                    tkn_hbm.at[idx_v.at[b + 1]], bufs[(b + 1) % 2], gsem
                ).wait()
        writes[B - 1].wait()

    return emb(x, tkn_table, pos_pk)


# speedup vs baseline: 1.0362x; 1.0362x over previous
import functools

import jax
import jax.numpy as jnp
from jax import lax
from jax.experimental import pallas as pl
from jax.experimental.pallas import tpu as pltpu
from jax.experimental.pallas import tpu_sc as plsc

_NUM_CORES = 2
_NUM_SUBCORES = 16
_LANES = 16


def kernel(x, tkn_table, pos_table):
    B, S = x.shape
    V, D = tkn_table.shape
    NW = _NUM_CORES * _NUM_SUBCORES
    C = S // NW
    H = C // 2
    NCH = B * 2
    assert S % NW == 0 and C % 2 == 0 and D % _LANES == 0

    x = x.astype(jnp.int32)

    mesh = plsc.VectorSubcoreMesh(core_axis_name="c", subcore_axis_name="s")

    @functools.partial(
        pl.kernel,
        mesh=mesh,
        out_type=jax.ShapeDtypeStruct((B, S, D), jnp.float32),
        scratch_types=[
            pltpu.VMEM((NCH, H), jnp.int32),
            pltpu.VMEM((C, D), jnp.float32),
            pltpu.VMEM((H, D), jnp.float32),
            pltpu.VMEM((H, D), jnp.float32),
            pltpu.SemaphoreType.DMA,
            pltpu.SemaphoreType.DMA,
            pltpu.SemaphoreType.DMA,
        ],
    )
    def emb(x_hbm, tkn_hbm, pos_hbm, out_hbm, idx_v, pos_v, t0, t1, gsem, w0, w1):
        bufs = [t0, t1]
        wsems = [w0, w1]
        wid = lax.axis_index("s") * _NUM_CORES + lax.axis_index("c")
        s0 = wid * C
        for k in range(NCH):
            b, h = divmod(k, 2)
            pltpu.sync_copy(x_hbm.at[b, pl.ds(s0 + h * H, H)], idx_v.at[k])
        pltpu.sync_copy(pos_hbm.at[pl.ds(s0, C)], pos_v)
        pltpu.async_copy(tkn_hbm.at[idx_v.at[0]], bufs[0], gsem).wait()

        writes = [None] * NCH
        for k in range(NCH):
            b, h = divmod(k, 2)
            cur = bufs[k % 2]
            if k >= 1:
                writes[k - 1].wait()

            def row_body(r, carry, _cur=cur, _h=h):
                for c in range(D // _LANES):
                    sl = pl.ds(c * _LANES, _LANES)
                    _cur[r, sl] = _cur[r, sl] + pos_v[_h * H + r, sl]
                return carry

            lax.fori_loop(0, H, row_body, 0)
            writes[k] = pltpu.async_copy(
                cur, out_hbm.at[b, pl.ds(s0 + h * H, H)], wsems[k % 2]
            )
            if k + 1 < NCH:
                pltpu.async_copy(
                    tkn_hbm.at[idx_v.at[k + 1]], bufs[(k + 1) % 2], gsem
                ).wait()
        writes[NCH - 1].wait()

    return emb(x, tkn_table, pos_table)


# speedup vs baseline: 1.4291x; 1.3791x over previous
import functools

import jax
import jax.numpy as jnp
from jax import lax
from jax.experimental import pallas as pl
from jax.experimental.pallas import tpu as pltpu
from jax.experimental.pallas import tpu_sc as plsc

_NUM_CORES = 2
_NUM_SUBCORES = 16
_LANES = 16


def kernel(x, tkn_table, pos_table):
    B, S = x.shape
    V, D = tkn_table.shape
    NW = _NUM_CORES * _NUM_SUBCORES
    C = S // NW
    assert S % NW == 0 and D % _LANES == 0

    x = x.astype(jnp.int32)

    mesh = plsc.VectorSubcoreMesh(core_axis_name="c", subcore_axis_name="s")

    @functools.partial(
        pl.kernel,
        mesh=mesh,
        out_type=jax.ShapeDtypeStruct((B, S, D), jnp.float32),
        scratch_types=[
            pltpu.VMEM((C,), jnp.int32),
            pltpu.VMEM((C, D), jnp.float32),
            pltpu.VMEM((C, D), jnp.float32),
            pltpu.SemaphoreType.DMA,
        ],
    )
    def emb(x_hbm, tkn_hbm, pos_hbm, out_hbm, idx_v, pos_v, tkn_v, sem):
        wid = lax.axis_index("s") * _NUM_CORES + lax.axis_index("c")
        s0 = wid * C
        pltpu.sync_copy(pos_hbm.at[pl.ds(s0, C)], pos_v)
        for b in range(B):
            pltpu.sync_copy(x_hbm.at[b, pl.ds(s0, C)], idx_v)
            pltpu.async_copy(tkn_hbm.at[idx_v], tkn_v, sem).wait()

            @plsc.parallel_loop(0, C, unroll=2)
            def row_body(r):
                for c in range(D // _LANES):
                    sl = pl.ds(c * _LANES, _LANES)
                    tkn_v[r, sl] = tkn_v[r, sl] + pos_v[r, sl]
            pltpu.sync_copy(tkn_v, out_hbm.at[b, pl.ds(s0, C)])

    return emb(x, tkn_table, pos_table)


# speedup vs baseline: 1.4653x; 1.0253x over previous
import functools

import jax
import jax.numpy as jnp
from jax import lax
from jax.experimental import pallas as pl
from jax.experimental.pallas import tpu as pltpu
from jax.experimental.pallas import tpu_sc as plsc

_NUM_CORES = 2
_NUM_SUBCORES = 16
_LANES = 16


def kernel(x, tkn_table, pos_table):
    B, S = x.shape
    V, D = tkn_table.shape
    NW = _NUM_CORES * _NUM_SUBCORES
    C = S // NW
    assert S % NW == 0 and D % _LANES == 0

    x = x.astype(jnp.int32)

    mesh = plsc.VectorSubcoreMesh(core_axis_name="c", subcore_axis_name="s")

    @functools.partial(
        pl.kernel,
        mesh=mesh,
        out_type=jax.ShapeDtypeStruct((B, S, D), jnp.float32),
        scratch_types=[
            pltpu.VMEM((C,), jnp.int32),
            pltpu.VMEM((C, D), jnp.float32),
            pltpu.VMEM((C, D), jnp.float32),
            pltpu.SemaphoreType.DMA,
        ],
    )
    def emb(x_hbm, tkn_hbm, pos_hbm, out_hbm, idx_v, pos_v, tkn_v, sem):
        wid = lax.axis_index("s") * _NUM_CORES + lax.axis_index("c")
        s0 = wid * C
        pltpu.sync_copy(pos_hbm.at[pl.ds(s0, C)], pos_v)
        for b in range(B):
            pltpu.sync_copy(x_hbm.at[b, pl.ds(s0, C)], idx_v)
            pltpu.async_copy(tkn_hbm.at[idx_v], tkn_v, sem).wait()

            def row_body(r, carry):
                for c in range(D // _LANES):
                    sl = pl.ds(c * _LANES, _LANES)
                    tkn_v[r, sl] = tkn_v[r, sl] + pos_v[r, sl]
                return carry

            lax.fori_loop(0, C, row_body, 0)
            pltpu.sync_copy(tkn_v, out_hbm.at[b, pl.ds(s0, C)])

    return emb(x, tkn_table, pos_table)
